# Initial kernel scaffold; baseline (speedup 1.0000x reference)
import jax
import jax.numpy as jnp
from jax.experimental import pallas as pl


def kernel(actions, states, ww, bw, wb, bb):
    raise NotImplementedError("write your pallas kernel here")


# speedup vs baseline: 3.7365x; 3.7365x over previous
import functools

import jax
import jax.numpy as jnp
from jax.experimental import pallas as pl
from jax.experimental.pallas import tpu as pltpu


def _hypermix_body(n_agents, action_dim, actions_ref, states_ref, ww_ref,
                   bw_ref, wb_ref, bb_ref, mixed_ref, b_ref):
    N, A = n_agents, action_dim
    states = states_ref[...]
    hw = jnp.dot(states, ww_ref[...],
                 preferred_element_type=jnp.float32) + bw_ref[...]
    b_ref[...] = jnp.dot(states, wb_ref[...],
                         preferred_element_type=jnp.float32) + bb_ref[...]
    acts = actions_ref[...]
    mixed = acts[:, 0:1] * jnp.abs(hw[:, 0:A])
    for n in range(1, N):
        mixed = mixed + acts[:, n:n + 1] * jnp.abs(hw[:, n * A:(n + 1) * A])
    mixed_ref[...] = mixed


def _broadcast_body(mixed_ref, b_ref, out_ref):
    out_ref[...] = b_ref[...] + mixed_ref[...][None, :, :]


def kernel(actions, states, ww, bw, wb, bb):
    f32 = jnp.float32
    actions = jnp.asarray(actions, f32)
    states = jnp.asarray(states, f32)
    B, N = actions.shape
    S = states.shape[1]
    NA = ww.shape[1]
    A = wb.shape[1]
    assert NA == N * A

    BB = 256 if B % 256 == 0 else B
    mixed, bvec = pl.pallas_call(
        functools.partial(_hypermix_body, N, A),
        grid=(B // BB,),
        in_specs=[
            pl.BlockSpec((BB, N), lambda i: (i, 0)),
            pl.BlockSpec((BB, S), lambda i: (i, 0)),
            pl.BlockSpec((S, NA), lambda i: (0, 0)),
            pl.BlockSpec((1, NA), lambda i: (0, 0)),
            pl.BlockSpec((S, A), lambda i: (0, 0)),
            pl.BlockSpec((1, A), lambda i: (0, 0)),
        ],
        out_specs=(pl.BlockSpec((BB, A), lambda i: (i, 0)),
                   pl.BlockSpec((BB, A), lambda i: (i, 0))),
        out_shape=(jax.ShapeDtypeStruct((B, A), f32),
                   jax.ShapeDtypeStruct((B, A), f32)),
        compiler_params=pltpu.CompilerParams(
            dimension_semantics=("parallel",)),
    )(actions, states, ww.astype(f32), bw.astype(f32),
      wb.astype(f32), bb.astype(f32))

    b3 = bvec.reshape(B, 1, A)
    TI = 16 if B % 16 == 0 else 8
    out = pl.pallas_call(
        _broadcast_body,
        grid=(B // TI,),
        in_specs=[
            pl.BlockSpec((B, A), lambda i: (0, 0)),
            pl.BlockSpec((TI, 1, A), lambda i: (i, 0, 0)),
        ],
        out_specs=pl.BlockSpec((TI, B, A), lambda i: (i, 0, 0)),
        out_shape=jax.ShapeDtypeStruct((B, B, A), f32),
        compiler_params=pltpu.CompilerParams(
            dimension_semantics=("parallel",)),
    )(mixed, b3)
    return out
